# baseline (speedup 1.0000x reference)
import functools

import jax
import jax.numpy as jnp
from jax import lax
from jax.experimental import pallas as pl
from jax.experimental.pallas import tpu as pltpu
from jax.experimental.pallas import tpu_sc as plsc

CHUNK = 128


def _sc_gather(dish2, store2, tags, tastes,
               dish_table, store_table, tag_table, taste_table):
  nc, ns = 2, 16
  nw = nc * ns
  nb = dish2.shape[0]
  B = nb * CHUNK
  assert nb % nw == 0
  cpw = nb // nw

  dd = dish_table.shape[1]
  ds_ = store_table.shape[1]
  de = tag_table.shape[1]

  mesh = plsc.VectorSubcoreMesh(core_axis_name="c", subcore_axis_name="s",
                                num_cores=nc, num_subcores=ns)

  @functools.partial(
      pl.kernel,
      mesh=mesh,
      out_type=jax.ShapeDtypeStruct((B, 128), jnp.float32),
      scratch_types=[
          pltpu.VMEM((CHUNK, 10), jnp.int32),
          pltpu.VMEM((CHUNK, 5), jnp.int32),
          pltpu.VMEM((17, CHUNK), jnp.int32),
          pltpu.VMEM((CHUNK, dd), jnp.float32),
          pltpu.VMEM((CHUNK, ds_), jnp.float32),
          pltpu.VMEM((10, CHUNK, de), jnp.float32),
          pltpu.VMEM((5, CHUNK, de), jnp.float32),
          pltpu.VMEM((CHUNK, de), jnp.float32),
          pltpu.VMEM((CHUNK, de), jnp.float32),
          pltpu.SemaphoreType.DMA,
          pltpu.SemaphoreType.DMA,
          pltpu.SemaphoreType.DMA,
      ],
      compiler_params=pltpu.CompilerParams(use_tc_tiling_on_sc=False,
                                           needs_layout_passes=False),
  )
  def gather_kernel(dish_i, store_i, tags_i, tastes_i,
                    dish_t, store_t, tag_t, taste_t,
                    a_o,
                    traw, sraw, idx_v, r_dish, r_store, r_tag, r_taste,
                    r_tsum, r_ssum,
                    sem_i, sem_g, sem_w):
    wid = lax.axis_index("s") * nc + lax.axis_index("c")
    iota16 = lax.iota(jnp.int32, 16)
    f32 = jnp.float32
    for c in range(cpw):
      r = wid * cpw + c
      base = r * CHUNK
      loads = [
          pltpu.async_copy(dish_i.at[r], idx_v.at[0], sem_i),
          pltpu.async_copy(store_i.at[r], idx_v.at[1], sem_i),
          pltpu.async_copy(tags_i.at[pl.ds(base, CHUNK), :], traw, sem_i),
          pltpu.async_copy(tastes_i.at[pl.ds(base, CHUNK), :], sraw, sem_i),
      ]
      for cp in loads:
        cp.wait()
      for v in range(CHUNK // 16):
        rows = iota16 + (v * 16)
        for j in range(10):
          idx_v[2 + j, pl.ds(v * 16, 16)] = plsc.load_gather(
              traw, [rows, jnp.full((16,), j, jnp.int32)])
        for j in range(5):
          idx_v[12 + j, pl.ds(v * 16, 16)] = plsc.load_gather(
              sraw, [rows, jnp.full((16,), j, jnp.int32)])
      gathers = [
          pltpu.async_copy(dish_t.at[idx_v.at[0]], r_dish, sem_g),
          pltpu.async_copy(store_t.at[idx_v.at[1]], r_store, sem_g),
      ]
      for j in range(10):
        gathers.append(
            pltpu.async_copy(tag_t.at[idx_v.at[2 + j]], r_tag.at[j], sem_g))
      for j in range(5):
        gathers.append(
            pltpu.async_copy(taste_t.at[idx_v.at[12 + j]], r_taste.at[j],
                             sem_g))
      for cp in gathers:
        cp.wait()

      def sum_body(v, carry):
        rows = iota16 + v * 16
        for d in range(de):
          cols = jnp.full((16,), d, jnp.int32)
          acc = plsc.load_gather(r_tag, [jnp.zeros((16,), jnp.int32),
                                         rows, cols])
          for j in range(1, 10):
            acc = acc + plsc.load_gather(
                r_tag, [jnp.full((16,), j, jnp.int32), rows, cols])
          plsc.store_scatter(r_tsum, [rows, cols], acc)
          acc2 = plsc.load_gather(r_taste, [jnp.zeros((16,), jnp.int32),
                                            rows, cols])
          for j in range(1, 5):
            acc2 = acc2 + plsc.load_gather(
                r_taste, [jnp.full((16,), j, jnp.int32), rows, cols])
          plsc.store_scatter(r_ssum, [rows, cols], acc2)
        return carry

      lax.fori_loop(0, CHUNK // 16, sum_body, 0)

      writes = [
          pltpu.async_copy(r_dish, a_o.at[pl.ds(base, CHUNK), pl.ds(0, dd)],
                           sem_w),
          pltpu.async_copy(r_store,
                           a_o.at[pl.ds(base, CHUNK), pl.ds(dd, ds_)], sem_w),
          pltpu.async_copy(r_tsum,
                           a_o.at[pl.ds(base, CHUNK), pl.ds(96, de)], sem_w),
          pltpu.async_copy(r_ssum,
                           a_o.at[pl.ds(base, CHUNK), pl.ds(112, de)], sem_w),
      ]
      for cp in writes:
        cp.wait()

  return gather_kernel(dish2, store2, tags, tastes,
                       dish_table, store_table, tag_table, taste_table)


def _tc_body(a_ref, tags_ref, tastes_ref, cat_ref, day_ref,
             price_ref, ot_ref, rt_ref, loc_ref, tm_ref,
             cat_t_ref,
             price_W_ref, price_b_ref, ot_W_ref, ot_b_ref,
             rt_W_ref, rt_b_ref, loc_W_ref, loc_b_ref, tm_W_ref, tm_b_ref,
             day_t_ref, W1_ref, b1_ref, W2_ref, b2_ref, W3_ref, b3_ref,
             out_ref):
  f32 = jnp.float32
  blk = a_ref.shape[0]

  tmask = (tags_ref[...] != 0).astype(f32)
  tcnt = jnp.sum(tmask, axis=1, keepdims=True)
  smask = (tastes_ref[...] != 0).astype(f32)
  scnt = jnp.sum(smask, axis=1, keepdims=True)
  rt_ = 1.0 / (tcnt + 1e-8)
  rs_ = 1.0 / (scnt + 1e-8)
  lane = lax.broadcasted_iota(jnp.int32, (blk, 128), 1)
  scale = jnp.where(lane < 96, 1.0, jnp.where(lane < 112, rt_, rs_))
  a = a_ref[...] * scale

  nc_ = cat_t_ref.shape[0]
  iota_c = lax.broadcasted_iota(jnp.int32, (blk, nc_), 1)
  cat_oh = (cat_ref[...] == iota_c).astype(f32)
  cat_emb = jnp.dot(cat_oh, cat_t_ref[...], preferred_element_type=f32)

  price_emb = price_ref[...] * price_W_ref[...] + price_b_ref[...]
  ot_emb = ot_ref[...] * ot_W_ref[...] + ot_b_ref[...]
  rt_emb = rt_ref[...] * rt_W_ref[...] + rt_b_ref[...]
  tm_emb = tm_ref[...] * tm_W_ref[...] + tm_b_ref[...]
  loc_emb = (jnp.dot(loc_ref[...], loc_W_ref[...],
                     preferred_element_type=f32) + loc_b_ref[...])

  iota7 = lax.broadcasted_iota(jnp.int32, (blk, 7), 1)
  day_oh = (day_ref[...] == iota7).astype(f32)
  day_emb = jnp.dot(day_oh, day_t_ref[...], preferred_element_type=f32)

  W1 = W1_ref[...]
  h = jnp.dot(a, W1[0:128, :], preferred_element_type=f32)
  h = h + jnp.dot(cat_emb, W1[128:144, :], preferred_element_type=f32)
  h = h + jnp.dot(price_emb, W1[144:160, :], preferred_element_type=f32)
  h = h + jnp.dot(ot_emb, W1[160:168, :], preferred_element_type=f32)
  h = h + jnp.dot(rt_emb, W1[168:176, :], preferred_element_type=f32)
  h = h + jnp.dot(loc_emb, W1[176:192, :], preferred_element_type=f32)
  h = h + jnp.dot(tm_emb, W1[192:200, :], preferred_element_type=f32)
  h = h + jnp.dot(day_emb, W1[200:208, :], preferred_element_type=f32)
  h = jnp.maximum(h + b1_ref[...], 0.0)

  h = jnp.dot(h, W2_ref[...], preferred_element_type=f32) + b2_ref[...]
  h = jnp.maximum(h, 0.0)
  out = jnp.dot(h, W3_ref[...], preferred_element_type=f32) + b3_ref[...]

  nrm = jnp.sqrt(jnp.sum(out * out, axis=-1, keepdims=True))
  out_ref[...] = out / jnp.maximum(nrm, 1e-12)


def kernel(dish_id, store_id, tags, tastes, category, price, order_times,
           rating, location, time_of_day, day_of_week, dish_table,
           store_table, tag_table, taste_table, cat_table, day_table,
           price_W, price_b, ot_W, ot_b, rt_W, rt_b, loc_W, loc_b, tm_W,
           tm_b, W1, b1, W2, b2, W3, b3):
  B = dish_id.shape[0]
  nb = B // CHUNK

  i32 = jnp.int32
  dish2 = dish_id.astype(i32).reshape(nb, CHUNK)
  store2 = store_id.astype(i32).reshape(nb, CHUNK)

  tag_tz = tag_table.at[0].set(0.0)
  taste_tz = taste_table.at[0].set(0.0)

  d_lin = jax.nn.relu(dish_table) - jax.nn.relu(-dish_table)
  a_packed = _sc_gather(dish2, store2, tags.astype(i32), tastes.astype(i32),
                        d_lin, store_table, tag_tz, taste_tz)

  BLK = 1024
  grid = (B // BLK,)

  def row_spec(w):
    return pl.BlockSpec((BLK, w), lambda i: (i, 0))

  def full_spec(shape):
    nd = len(shape)
    return pl.BlockSpec(shape, lambda i: (0,) * nd)

  out = pl.pallas_call(
      _tc_body,
      grid=grid,
      in_specs=[
          row_spec(128),
          row_spec(10), row_spec(5), row_spec(1), row_spec(1),
          row_spec(1), row_spec(1), row_spec(1), row_spec(2), row_spec(1),
          full_spec((1000, 16)),
          full_spec((1, 16)), full_spec((1, 16)),
          full_spec((1, 8)), full_spec((1, 8)),
          full_spec((1, 8)), full_spec((1, 8)),
          full_spec((2, 16)), full_spec((1, 16)),
          full_spec((1, 8)), full_spec((1, 8)),
          full_spec((7, 8)),
          full_spec((208, 128)), full_spec((1, 128)),
          full_spec((128, 64)), full_spec((1, 64)),
          full_spec((64, 64)), full_spec((1, 64)),
      ],
      out_specs=row_spec(64),
      out_shape=jax.ShapeDtypeStruct((B, 64), jnp.float32),
      compiler_params=pltpu.CompilerParams(
          dimension_semantics=("parallel",)),
  )(
      a_packed,
      tags.astype(i32), tastes.astype(i32),
      category.astype(i32).reshape(B, 1),
      day_of_week.astype(i32).reshape(B, 1),
      price, order_times, rating, location, time_of_day,
      cat_table,
      price_W, price_b.reshape(1, 16), ot_W, ot_b.reshape(1, 8),
      rt_W, rt_b.reshape(1, 8), loc_W, loc_b.reshape(1, 16),
      tm_W, tm_b.reshape(1, 8), day_table,
      W1, b1.reshape(1, 128), W2, b2.reshape(1, 64),
      W3, b3.reshape(1, 64),
  )
  return out


# speedup vs baseline: 1.4474x; 1.4474x over previous
import functools

import jax
import jax.numpy as jnp
from jax import lax
from jax.experimental import pallas as pl
from jax.experimental.pallas import tpu as pltpu
from jax.experimental.pallas import tpu_sc as plsc

CHUNK = 128


def _sc_dish(dish2, dish_table, zpad):
  nc, ns = 2, 16
  nw = nc * ns
  nb = dish2.shape[0]
  B = nb * CHUNK
  cpw = nb // nw
  dd = dish_table.shape[1]

  mesh = plsc.VectorSubcoreMesh(core_axis_name="c", subcore_axis_name="s",
                                num_cores=nc, num_subcores=ns)

  @functools.partial(
      pl.kernel,
      mesh=mesh,
      out_type=jax.ShapeDtypeStruct((B, 128), jnp.float32),
      scratch_types=[
          pltpu.VMEM((CHUNK,), jnp.int32),
          pltpu.VMEM((CHUNK, dd), jnp.float32),
          pltpu.VMEM((CHUNK, 64), jnp.float32),
          pltpu.SemaphoreType.DMA,
          pltpu.SemaphoreType.DMA,
          pltpu.SemaphoreType.DMA,
      ],
      compiler_params=pltpu.CompilerParams(use_tc_tiling_on_sc=False,
                                           needs_layout_passes=False),
  )
  def k1(dish_i, dish_t, zpad_i, a1_o, idxd, r_dish, zbuf,
         sem_i, sem_g, sem_w):
    wid = lax.axis_index("s") * nc + lax.axis_index("c")
    pltpu.async_copy(zpad_i, zbuf, sem_i).wait()
    for c in range(cpw):
      r = wid * cpw + c
      base = r * CHUNK
      pltpu.async_copy(dish_i.at[r], idxd, sem_i).wait()
      pltpu.async_copy(dish_t.at[idxd], r_dish, sem_g).wait()
      w1 = pltpu.async_copy(
          r_dish, a1_o.at[pl.ds(base, CHUNK), pl.ds(0, dd)], sem_w)
      w2 = pltpu.async_copy(
          zbuf, a1_o.at[pl.ds(base, CHUNK), pl.ds(dd, 64)], sem_w)
      w1.wait()
      w2.wait()

  return k1(dish2, dish_table, zpad)


def _sc_gather(store2, tags, tastes, store_table, tag_table, taste_table,
               zpad):
  nc, ns = 2, 16
  nw = nc * ns
  nb = store2.shape[0]
  B = nb * CHUNK
  assert nb % nw == 0
  cpw = nb // nw

  ds_ = store_table.shape[1]
  de = tag_table.shape[1]

  mesh = plsc.VectorSubcoreMesh(core_axis_name="c", subcore_axis_name="s",
                                num_cores=nc, num_subcores=ns)

  @functools.partial(
      pl.kernel,
      mesh=mesh,
      out_type=jax.ShapeDtypeStruct((B, 128), jnp.float32),
      scratch_types=[
          pltpu.VMEM((CHUNK, 10), jnp.int32),
          pltpu.VMEM((CHUNK, 5), jnp.int32),
          pltpu.VMEM((16, CHUNK), jnp.int32),
          pltpu.VMEM((CHUNK, ds_), jnp.float32),
          pltpu.VMEM((10, CHUNK, de), jnp.float32),
          pltpu.VMEM((5, CHUNK, de), jnp.float32),
          pltpu.VMEM((CHUNK, de), jnp.float32),
          pltpu.VMEM((CHUNK, de), jnp.float32),
          pltpu.VMEM((CHUNK, 64), jnp.float32),
          pltpu.SemaphoreType.DMA,
          pltpu.SemaphoreType.DMA,
          pltpu.SemaphoreType.DMA,
      ],
      compiler_params=pltpu.CompilerParams(use_tc_tiling_on_sc=False,
                                           needs_layout_passes=False),
  )
  def gather_kernel(store_i, tags_i, tastes_i,
                    store_t, tag_t, taste_t, zpad_i,
                    a_o,
                    traw, sraw, idx_v, r_store, r_tag, r_taste,
                    r_tsum, r_ssum, zbuf,
                    sem_i, sem_g, sem_w):
    wid = lax.axis_index("s") * nc + lax.axis_index("c")
    iota16 = lax.iota(jnp.int32, 16)
    pltpu.async_copy(zpad_i, zbuf, sem_i).wait()
    for c in range(cpw):
      r = wid * cpw + c
      base = r * CHUNK
      loads = [
          pltpu.async_copy(store_i.at[r], idx_v.at[0], sem_i),
          pltpu.async_copy(tags_i.at[pl.ds(base, CHUNK), :], traw, sem_i),
          pltpu.async_copy(tastes_i.at[pl.ds(base, CHUNK), :], sraw, sem_i),
      ]
      for cp in loads:
        cp.wait()
      for v in range(CHUNK // 16):
        rows = iota16 + (v * 16)
        for j in range(10):
          idx_v[1 + j, pl.ds(v * 16, 16)] = plsc.load_gather(
              traw, [rows, jnp.full((16,), j, jnp.int32)])
        for j in range(5):
          idx_v[11 + j, pl.ds(v * 16, 16)] = plsc.load_gather(
              sraw, [rows, jnp.full((16,), j, jnp.int32)])
      gathers = [
          pltpu.async_copy(store_t.at[idx_v.at[0]], r_store, sem_g),
      ]
      for j in range(10):
        gathers.append(
            pltpu.async_copy(tag_t.at[idx_v.at[1 + j]], r_tag.at[j], sem_g))
      for j in range(5):
        gathers.append(
            pltpu.async_copy(taste_t.at[idx_v.at[11 + j]], r_taste.at[j],
                             sem_g))
      for cp in gathers:
        cp.wait()

      def sum_body(v, carry):
        rows = iota16 + v * 16
        for d in range(de):
          cols = jnp.full((16,), d, jnp.int32)
          acc = plsc.load_gather(r_tag, [jnp.zeros((16,), jnp.int32),
                                         rows, cols])
          for j in range(1, 10):
            acc = acc + plsc.load_gather(
                r_tag, [jnp.full((16,), j, jnp.int32), rows, cols])
          plsc.store_scatter(r_tsum, [rows, cols], acc)
          acc2 = plsc.load_gather(r_taste, [jnp.zeros((16,), jnp.int32),
                                            rows, cols])
          for j in range(1, 5):
            acc2 = acc2 + plsc.load_gather(
                r_taste, [jnp.full((16,), j, jnp.int32), rows, cols])
          plsc.store_scatter(r_ssum, [rows, cols], acc2)
        return carry

      lax.fori_loop(0, CHUNK // 16, sum_body, 0)

      writes = [
          pltpu.async_copy(r_store,
                           a_o.at[pl.ds(base, CHUNK), pl.ds(0, ds_)], sem_w),
          pltpu.async_copy(r_tsum,
                           a_o.at[pl.ds(base, CHUNK), pl.ds(32, de)], sem_w),
          pltpu.async_copy(r_ssum,
                           a_o.at[pl.ds(base, CHUNK), pl.ds(48, de)], sem_w),
          pltpu.async_copy(zbuf,
                           a_o.at[pl.ds(base, CHUNK), pl.ds(64, 64)], sem_w),
      ]
      for cp in writes:
        cp.wait()

  return gather_kernel(store2, tags, tastes,
                       store_table, tag_table, taste_table, zpad)


def _tc_body(a1_ref, a2_ref, tags_ref, tastes_ref, cat_ref, day_ref,
             price_ref, ot_ref, rt_ref, loc_ref, tm_ref,
             cat_t_ref,
             price_W_ref, price_b_ref, ot_W_ref, ot_b_ref,
             rt_W_ref, rt_b_ref, loc_W_ref, loc_b_ref, tm_W_ref, tm_b_ref,
             day_t_ref, W1a_ref, W1b_ref, b1_ref, W2_ref, b2_ref, W3_ref,
             b3_ref, W1r_ref, out_ref):
  f32 = jnp.float32
  blk = a1_ref.shape[0]

  tmask = (tags_ref[...] != 0).astype(f32)
  tcnt = jnp.sum(tmask, axis=1, keepdims=True)
  smask = (tastes_ref[...] != 0).astype(f32)
  scnt = jnp.sum(smask, axis=1, keepdims=True)
  rt_ = 1.0 / (tcnt + 1e-8)
  rs_ = 1.0 / (scnt + 1e-8)
  lane = lax.broadcasted_iota(jnp.int32, (blk, 128), 1)
  scale = jnp.where(lane < 32, 1.0,
                    jnp.where(lane < 48, rt_,
                              jnp.where(lane < 64, rs_, 1.0)))
  a2 = a2_ref[...] * scale

  nc_ = cat_t_ref.shape[0]
  iota_c = lax.broadcasted_iota(jnp.int32, (blk, nc_), 1)
  cat_oh = (cat_ref[...] == iota_c).astype(f32)
  cat_emb = jnp.dot(cat_oh, cat_t_ref[...], preferred_element_type=f32)

  price_emb = price_ref[...] * price_W_ref[...] + price_b_ref[...]
  ot_emb = ot_ref[...] * ot_W_ref[...] + ot_b_ref[...]
  rt_emb = rt_ref[...] * rt_W_ref[...] + rt_b_ref[...]
  tm_emb = tm_ref[...] * tm_W_ref[...] + tm_b_ref[...]
  loc_emb = (jnp.dot(loc_ref[...], loc_W_ref[...],
                     preferred_element_type=f32) + loc_b_ref[...])

  iota7 = lax.broadcasted_iota(jnp.int32, (blk, 7), 1)
  day_oh = (day_ref[...] == iota7).astype(f32)
  day_emb = jnp.dot(day_oh, day_t_ref[...], preferred_element_type=f32)

  W1r = W1r_ref[...]
  h = jnp.dot(a1_ref[...], W1a_ref[...], preferred_element_type=f32)
  h = h + jnp.dot(a2, W1b_ref[...], preferred_element_type=f32)
  h = h + jnp.dot(cat_emb, W1r[0:16, :], preferred_element_type=f32)
  h = h + jnp.dot(price_emb, W1r[16:32, :], preferred_element_type=f32)
  h = h + jnp.dot(ot_emb, W1r[32:40, :], preferred_element_type=f32)
  h = h + jnp.dot(rt_emb, W1r[40:48, :], preferred_element_type=f32)
  h = h + jnp.dot(loc_emb, W1r[48:64, :], preferred_element_type=f32)
  h = h + jnp.dot(tm_emb, W1r[64:72, :], preferred_element_type=f32)
  h = h + jnp.dot(day_emb, W1r[72:80, :], preferred_element_type=f32)
  h = jnp.maximum(h + b1_ref[...], 0.0)

  h = jnp.dot(h, W2_ref[...], preferred_element_type=f32) + b2_ref[...]
  h = jnp.maximum(h, 0.0)
  out = jnp.dot(h, W3_ref[...], preferred_element_type=f32) + b3_ref[...]

  nrm = jnp.sqrt(jnp.sum(out * out, axis=-1, keepdims=True))
  out_ref[...] = out / jnp.maximum(nrm, 1e-12)


def kernel(dish_id, store_id, tags, tastes, category, price, order_times,
           rating, location, time_of_day, day_of_week, dish_table,
           store_table, tag_table, taste_table, cat_table, day_table,
           price_W, price_b, ot_W, ot_b, rt_W, rt_b, loc_W, loc_b, tm_W,
           tm_b, W1, b1, W2, b2, W3, b3):
  B = dish_id.shape[0]
  nb = B // CHUNK

  i32 = jnp.int32
  dish2 = dish_id.astype(i32).reshape(nb, CHUNK)
  store2 = store_id.astype(i32).reshape(nb, CHUNK)

  tag_tz = tag_table.at[0].set(0.0)
  taste_tz = taste_table.at[0].set(0.0)

  zpad = jnp.zeros((CHUNK, 64), jnp.float32)
  a1 = _sc_dish(dish2, dish_table, zpad)
  a2 = _sc_gather(store2, tags.astype(i32), tastes.astype(i32),
                  store_table, tag_tz, taste_tz, zpad)

  zrows = jnp.zeros((64, 128), jnp.float32)
  W1a = jnp.concatenate([W1[0:64], zrows], 0)
  W1b = jnp.concatenate([W1[64:128], zrows], 0)

  BLK = 1024
  grid = (B // BLK,)

  def row_spec(w):
    return pl.BlockSpec((BLK, w), lambda i: (i, 0))

  def full_spec(shape):
    nd = len(shape)
    return pl.BlockSpec(shape, lambda i: (0,) * nd)

  out = pl.pallas_call(
      _tc_body,
      grid=grid,
      in_specs=[
          row_spec(128), row_spec(128),
          row_spec(10), row_spec(5), row_spec(1), row_spec(1),
          row_spec(1), row_spec(1), row_spec(1), row_spec(2), row_spec(1),
          full_spec((1000, 16)),
          full_spec((1, 16)), full_spec((1, 16)),
          full_spec((1, 8)), full_spec((1, 8)),
          full_spec((1, 8)), full_spec((1, 8)),
          full_spec((2, 16)), full_spec((1, 16)),
          full_spec((1, 8)), full_spec((1, 8)),
          full_spec((7, 8)),
          full_spec((128, 128)), full_spec((128, 128)),
          full_spec((1, 128)),
          full_spec((128, 64)), full_spec((1, 64)),
          full_spec((64, 64)), full_spec((1, 64)),
          full_spec((80, 128)),
      ],
      out_specs=row_spec(64),
      out_shape=jax.ShapeDtypeStruct((B, 64), jnp.float32),
      compiler_params=pltpu.CompilerParams(
          dimension_semantics=("parallel",)),
  )(
      a1, a2,
      tags.astype(i32), tastes.astype(i32),
      category.astype(i32).reshape(B, 1),
      day_of_week.astype(i32).reshape(B, 1),
      price, order_times, rating, location, time_of_day,
      cat_table,
      price_W, price_b.reshape(1, 16), ot_W, ot_b.reshape(1, 8),
      rt_W, rt_b.reshape(1, 8), loc_W, loc_b.reshape(1, 16),
      tm_W, tm_b.reshape(1, 8), day_table,
      W1a, W1b, b1.reshape(1, 128), W2, b2.reshape(1, 64),
      W3, b3.reshape(1, 64), W1[128:208],
  )
  return out


# speedup vs baseline: 1.4680x; 1.0142x over previous
import functools

import jax
import jax.numpy as jnp
from jax import lax
from jax.experimental import pallas as pl
from jax.experimental.pallas import tpu as pltpu
from jax.experimental.pallas import tpu_sc as plsc

CHUNK = 128


def _sc_dish(dish2, dish_table, zpad):
  nc, ns = 2, 16
  nw = nc * ns
  nb = dish2.shape[0]
  B = nb * CHUNK
  cpw = nb // nw
  dd = dish_table.shape[1]

  mesh = plsc.VectorSubcoreMesh(core_axis_name="c", subcore_axis_name="s",
                                num_cores=nc, num_subcores=ns)

  @functools.partial(
      pl.kernel,
      mesh=mesh,
      out_type=jax.ShapeDtypeStruct((B, 128), jnp.float32),
      scratch_types=[
          pltpu.VMEM((4, CHUNK), jnp.int32),
          pltpu.VMEM((4, CHUNK, dd), jnp.float32),
          pltpu.VMEM((CHUNK, 64), jnp.float32),
          pltpu.SemaphoreType.DMA,
          pltpu.SemaphoreType.DMA,
          pltpu.SemaphoreType.DMA,
      ],
      compiler_params=pltpu.CompilerParams(use_tc_tiling_on_sc=False,
                                           needs_layout_passes=False),
  )
  def k1(dish_i, dish_t, zpad_i, a1_o, idxd, r_dish, zbuf,
         sem_i, sem_g, sem_w):
    wid = lax.axis_index("s") * nc + lax.axis_index("c")
    zc = pltpu.async_copy(zpad_i, zbuf, sem_i)
    loads = [pltpu.async_copy(dish_i.at[wid * cpw + c], idxd.at[c], sem_i)
             for c in range(cpw)]
    zc.wait()
    for cp in loads:
      cp.wait()
    gathers = [pltpu.async_copy(dish_t.at[idxd.at[c]], r_dish.at[c], sem_g)
               for c in range(cpw)]
    writes = []
    for c in range(cpw):
      gathers[c].wait()
      base = (wid * cpw + c) * CHUNK
      writes.append(pltpu.async_copy(
          r_dish.at[c], a1_o.at[pl.ds(base, CHUNK), pl.ds(0, dd)], sem_w))
      writes.append(pltpu.async_copy(
          zbuf, a1_o.at[pl.ds(base, CHUNK), pl.ds(dd, 64)], sem_w))
    for cp in writes:
      cp.wait()

  return k1(dish2, dish_table, zpad)


def _sc_gather(store2, cat2, tags, tastes, store_table, cat_table,
               tag_table, taste_table, zpad):
  nc, ns = 2, 16
  nw = nc * ns
  nb = store2.shape[0]
  B = nb * CHUNK
  assert nb % nw == 0
  cpw = nb // nw

  ds_ = store_table.shape[1]
  de = tag_table.shape[1]

  mesh = plsc.VectorSubcoreMesh(core_axis_name="c", subcore_axis_name="s",
                                num_cores=nc, num_subcores=ns)

  @functools.partial(
      pl.kernel,
      mesh=mesh,
      out_type=jax.ShapeDtypeStruct((B, 128), jnp.float32),
      scratch_types=[
          pltpu.VMEM((CHUNK, 10), jnp.int32),
          pltpu.VMEM((CHUNK, 5), jnp.int32),
          pltpu.VMEM((17, CHUNK), jnp.int32),
          pltpu.VMEM((CHUNK, ds_), jnp.float32),
          pltpu.VMEM((CHUNK, de), jnp.float32),
          pltpu.VMEM((10, CHUNK, de), jnp.float32),
          pltpu.VMEM((5, CHUNK, de), jnp.float32),
          pltpu.VMEM((CHUNK, de), jnp.float32),
          pltpu.VMEM((CHUNK, de), jnp.float32),
          pltpu.VMEM((CHUNK, 48), jnp.float32),
          pltpu.SemaphoreType.DMA,
          pltpu.SemaphoreType.DMA,
          pltpu.SemaphoreType.DMA,
      ],
      compiler_params=pltpu.CompilerParams(use_tc_tiling_on_sc=False,
                                           needs_layout_passes=False),
  )
  def gather_kernel(store_i, cat_i, tags_i, tastes_i,
                    store_t, cat_t, tag_t, taste_t, zpad_i,
                    a_o,
                    traw, sraw, idx_v, r_store, r_cat, r_tag, r_taste,
                    r_tsum, r_ssum, zbuf,
                    sem_i, sem_g, sem_w):
    wid = lax.axis_index("s") * nc + lax.axis_index("c")
    iota16 = lax.iota(jnp.int32, 16)
    pltpu.async_copy(zpad_i, zbuf, sem_i).wait()
    for c in range(cpw):
      r = wid * cpw + c
      base = r * CHUNK
      loads = [
          pltpu.async_copy(store_i.at[r], idx_v.at[0], sem_i),
          pltpu.async_copy(cat_i.at[r], idx_v.at[16], sem_i),
          pltpu.async_copy(tags_i.at[pl.ds(base, CHUNK), :], traw, sem_i),
          pltpu.async_copy(tastes_i.at[pl.ds(base, CHUNK), :], sraw, sem_i),
      ]
      for cp in loads:
        cp.wait()
      for v in range(CHUNK // 16):
        rows = iota16 + (v * 16)
        for j in range(10):
          idx_v[1 + j, pl.ds(v * 16, 16)] = plsc.load_gather(
              traw, [rows, jnp.full((16,), j, jnp.int32)])
        for j in range(5):
          idx_v[11 + j, pl.ds(v * 16, 16)] = plsc.load_gather(
              sraw, [rows, jnp.full((16,), j, jnp.int32)])
      gathers = [
          pltpu.async_copy(store_t.at[idx_v.at[0]], r_store, sem_g),
          pltpu.async_copy(cat_t.at[idx_v.at[16]], r_cat, sem_g),
      ]
      for j in range(10):
        gathers.append(
            pltpu.async_copy(tag_t.at[idx_v.at[1 + j]], r_tag.at[j], sem_g))
      for j in range(5):
        gathers.append(
            pltpu.async_copy(taste_t.at[idx_v.at[11 + j]], r_taste.at[j],
                             sem_g))
      for cp in gathers:
        cp.wait()

      def sum_body(v, carry):
        rows = iota16 + v * 16
        for d in range(de):
          cols = jnp.full((16,), d, jnp.int32)
          acc = plsc.load_gather(r_tag, [jnp.zeros((16,), jnp.int32),
                                         rows, cols])
          for j in range(1, 10):
            acc = acc + plsc.load_gather(
                r_tag, [jnp.full((16,), j, jnp.int32), rows, cols])
          plsc.store_scatter(r_tsum, [rows, cols], acc)
          acc2 = plsc.load_gather(r_taste, [jnp.zeros((16,), jnp.int32),
                                            rows, cols])
          for j in range(1, 5):
            acc2 = acc2 + plsc.load_gather(
                r_taste, [jnp.full((16,), j, jnp.int32), rows, cols])
          plsc.store_scatter(r_ssum, [rows, cols], acc2)
        return carry

      lax.fori_loop(0, CHUNK // 16, sum_body, 0)

      writes = [
          pltpu.async_copy(r_store,
                           a_o.at[pl.ds(base, CHUNK), pl.ds(0, ds_)], sem_w),
          pltpu.async_copy(r_tsum,
                           a_o.at[pl.ds(base, CHUNK), pl.ds(32, de)], sem_w),
          pltpu.async_copy(r_ssum,
                           a_o.at[pl.ds(base, CHUNK), pl.ds(48, de)], sem_w),
          pltpu.async_copy(r_cat,
                           a_o.at[pl.ds(base, CHUNK), pl.ds(64, de)], sem_w),
          pltpu.async_copy(zbuf,
                           a_o.at[pl.ds(base, CHUNK), pl.ds(80, 48)], sem_w),
      ]
      for cp in writes:
        cp.wait()

  return gather_kernel(store2, cat2, tags, tastes,
                       store_table, cat_table, tag_table, taste_table, zpad)


def _tc_body(a1_ref, a2_ref, tags_ref, tastes_ref, day_ref,
             price_ref, ot_ref, rt_ref, loc_ref, tm_ref,
             price_W_ref, price_b_ref, ot_W_ref, ot_b_ref,
             rt_W_ref, rt_b_ref, loc_W_ref, loc_b_ref, tm_W_ref, tm_b_ref,
             day_t_ref, W1a_ref, W1b_ref, b1_ref, W2_ref, b2_ref, W3_ref,
             b3_ref, W1r_ref, out_ref):
  f32 = jnp.float32
  blk = a1_ref.shape[0]

  tmask = (tags_ref[...] != 0).astype(f32)
  tcnt = jnp.sum(tmask, axis=1, keepdims=True)
  smask = (tastes_ref[...] != 0).astype(f32)
  scnt = jnp.sum(smask, axis=1, keepdims=True)
  rt_ = 1.0 / (tcnt + 1e-8)
  rs_ = 1.0 / (scnt + 1e-8)
  lane = lax.broadcasted_iota(jnp.int32, (blk, 128), 1)
  scale = jnp.where(lane < 32, 1.0,
                    jnp.where(lane < 48, rt_,
                              jnp.where(lane < 64, rs_, 1.0)))
  a2 = a2_ref[...] * scale

  price_emb = price_ref[...] * price_W_ref[...] + price_b_ref[...]
  ot_emb = ot_ref[...] * ot_W_ref[...] + ot_b_ref[...]
  rt_emb = rt_ref[...] * rt_W_ref[...] + rt_b_ref[...]
  tm_emb = tm_ref[...] * tm_W_ref[...] + tm_b_ref[...]
  loc_emb = (jnp.dot(loc_ref[...], loc_W_ref[...],
                     preferred_element_type=f32) + loc_b_ref[...])

  iota7 = lax.broadcasted_iota(jnp.int32, (blk, 7), 1)
  day_oh = (day_ref[...] == iota7).astype(f32)
  day_emb = jnp.dot(day_oh, day_t_ref[...], preferred_element_type=f32)

  W1r = W1r_ref[...]
  h = jnp.dot(a1_ref[...], W1a_ref[...], preferred_element_type=f32)
  h = h + jnp.dot(a2, W1b_ref[...], preferred_element_type=f32)
  h = h + jnp.dot(price_emb, W1r[0:16, :], preferred_element_type=f32)
  h = h + jnp.dot(ot_emb, W1r[16:24, :], preferred_element_type=f32)
  h = h + jnp.dot(rt_emb, W1r[24:32, :], preferred_element_type=f32)
  h = h + jnp.dot(loc_emb, W1r[32:48, :], preferred_element_type=f32)
  h = h + jnp.dot(tm_emb, W1r[48:56, :], preferred_element_type=f32)
  h = h + jnp.dot(day_emb, W1r[56:64, :], preferred_element_type=f32)
  h = jnp.maximum(h + b1_ref[...], 0.0)

  h = jnp.dot(h, W2_ref[...], preferred_element_type=f32) + b2_ref[...]
  h = jnp.maximum(h, 0.0)
  out = jnp.dot(h, W3_ref[...], preferred_element_type=f32) + b3_ref[...]

  nrm = jnp.sqrt(jnp.sum(out * out, axis=-1, keepdims=True))
  out_ref[...] = out / jnp.maximum(nrm, 1e-12)


def kernel(dish_id, store_id, tags, tastes, category, price, order_times,
           rating, location, time_of_day, day_of_week, dish_table,
           store_table, tag_table, taste_table, cat_table, day_table,
           price_W, price_b, ot_W, ot_b, rt_W, rt_b, loc_W, loc_b, tm_W,
           tm_b, W1, b1, W2, b2, W3, b3):
  B = dish_id.shape[0]
  nb = B // CHUNK

  i32 = jnp.int32
  dish2 = dish_id.astype(i32).reshape(nb, CHUNK)
  store2 = store_id.astype(i32).reshape(nb, CHUNK)

  tag_tz = tag_table.at[0].set(0.0)
  taste_tz = taste_table.at[0].set(0.0)

  zpad = jnp.zeros((CHUNK, 64), jnp.float32)
  zpad2 = jnp.zeros((CHUNK, 48), jnp.float32)
  cat2 = category.astype(i32).reshape(nb, CHUNK)
  a1 = _sc_dish(dish2, dish_table, zpad)
  a2 = _sc_gather(store2, cat2, tags.astype(i32), tastes.astype(i32),
                  store_table, cat_table, tag_tz, taste_tz, zpad2)

  W1a = jnp.concatenate([W1[0:64], jnp.zeros((64, 128), jnp.float32)], 0)
  W1b = jnp.concatenate([W1[64:144], jnp.zeros((48, 128), jnp.float32)], 0)

  BLK = 1024
  grid = (B // BLK,)

  def row_spec(w):
    return pl.BlockSpec((BLK, w), lambda i: (i, 0))

  def full_spec(shape):
    nd = len(shape)
    return pl.BlockSpec(shape, lambda i: (0,) * nd)

  out = pl.pallas_call(
      _tc_body,
      grid=grid,
      in_specs=[
          row_spec(128), row_spec(128),
          row_spec(10), row_spec(5), row_spec(1),
          row_spec(1), row_spec(1), row_spec(1), row_spec(2), row_spec(1),
          full_spec((1, 16)), full_spec((1, 16)),
          full_spec((1, 8)), full_spec((1, 8)),
          full_spec((1, 8)), full_spec((1, 8)),
          full_spec((2, 16)), full_spec((1, 16)),
          full_spec((1, 8)), full_spec((1, 8)),
          full_spec((7, 8)),
          full_spec((128, 128)), full_spec((128, 128)),
          full_spec((1, 128)),
          full_spec((128, 64)), full_spec((1, 64)),
          full_spec((64, 64)), full_spec((1, 64)),
          full_spec((64, 128)),
      ],
      out_specs=row_spec(64),
      out_shape=jax.ShapeDtypeStruct((B, 64), jnp.float32),
      compiler_params=pltpu.CompilerParams(
          dimension_semantics=("parallel",)),
  )(
      a1, a2,
      tags.astype(i32), tastes.astype(i32),
      day_of_week.astype(i32).reshape(B, 1),
      price, order_times, rating, location, time_of_day,
      price_W, price_b.reshape(1, 16), ot_W, ot_b.reshape(1, 8),
      rt_W, rt_b.reshape(1, 8), loc_W, loc_b.reshape(1, 16),
      tm_W, tm_b.reshape(1, 8), day_table,
      W1a, W1b, b1.reshape(1, 128), W2, b2.reshape(1, 64),
      W3, b3.reshape(1, 64), W1[144:208],
  )
  return out


# speedup vs baseline: 1.4786x; 1.0072x over previous
import functools

import jax
import jax.numpy as jnp
from jax import lax
from jax.experimental import pallas as pl
from jax.experimental.pallas import tpu as pltpu
from jax.experimental.pallas import tpu_sc as plsc

CHUNK = 128


def _sc_dish(dish2, dish_table, zpad):
  nc, ns = 2, 16
  nw = nc * ns
  nb = dish2.shape[0]
  B = nb * CHUNK
  cpw = nb // nw
  dd = dish_table.shape[1]

  mesh = plsc.VectorSubcoreMesh(core_axis_name="c", subcore_axis_name="s",
                                num_cores=nc, num_subcores=ns)

  @functools.partial(
      pl.kernel,
      mesh=mesh,
      out_type=jax.ShapeDtypeStruct((B, 128), jnp.float32),
      scratch_types=[
          pltpu.VMEM((4, CHUNK), jnp.int32),
          pltpu.VMEM((4, CHUNK, dd), jnp.float32),
          pltpu.VMEM((CHUNK, 64), jnp.float32),
          pltpu.SemaphoreType.DMA,
          pltpu.SemaphoreType.DMA,
          pltpu.SemaphoreType.DMA,
      ],
      compiler_params=pltpu.CompilerParams(use_tc_tiling_on_sc=False,
                                           needs_layout_passes=False),
  )
  def k1(dish_i, dish_t, zpad_i, a1_o, idxd, r_dish, zbuf,
         sem_i, sem_g, sem_w):
    wid = lax.axis_index("s") * nc + lax.axis_index("c")
    zc = pltpu.async_copy(zpad_i, zbuf, sem_i)
    loads = [pltpu.async_copy(dish_i.at[wid * cpw + c], idxd.at[c], sem_i)
             for c in range(cpw)]
    zc.wait()
    for cp in loads:
      cp.wait()
    gathers = [pltpu.async_copy(dish_t.at[idxd.at[c]], r_dish.at[c], sem_g)
               for c in range(cpw)]
    writes = []
    for c in range(cpw):
      gathers[c].wait()
      base = (wid * cpw + c) * CHUNK
      writes.append(pltpu.async_copy(
          r_dish.at[c], a1_o.at[pl.ds(base, CHUNK), pl.ds(0, dd)], sem_w))
      writes.append(pltpu.async_copy(
          zbuf, a1_o.at[pl.ds(base, CHUNK), pl.ds(dd, 64)], sem_w))
    for cp in writes:
      cp.wait()

  return k1(dish2, dish_table, zpad)


def _sc_gather(store2, cat2, tags, tastes, store_table, cat_table,
               tag_table, taste_table, zpad):
  nc, ns = 2, 16
  nw = nc * ns
  nb = store2.shape[0]
  B = nb * CHUNK
  assert nb % nw == 0
  cpw = nb // nw

  ds_ = store_table.shape[1]
  de = tag_table.shape[1]

  mesh = plsc.VectorSubcoreMesh(core_axis_name="c", subcore_axis_name="s",
                                num_cores=nc, num_subcores=ns)

  @functools.partial(
      pl.kernel,
      mesh=mesh,
      out_type=jax.ShapeDtypeStruct((B, 128), jnp.float32),
      scratch_types=[
          pltpu.VMEM((CHUNK, 10), jnp.int32),
          pltpu.VMEM((CHUNK, 5), jnp.int32),
          pltpu.VMEM((17, CHUNK), jnp.int32),
          pltpu.VMEM((CHUNK, ds_), jnp.float32),
          pltpu.VMEM((CHUNK, de), jnp.float32),
          pltpu.VMEM((10, CHUNK, de), jnp.float32),
          pltpu.VMEM((5, CHUNK, de), jnp.float32),
          pltpu.VMEM((CHUNK, de), jnp.float32),
          pltpu.VMEM((CHUNK, de), jnp.float32),
          pltpu.VMEM((CHUNK, 48), jnp.float32),
          pltpu.SemaphoreType.DMA,
          pltpu.SemaphoreType.DMA,
          pltpu.SemaphoreType.DMA,
      ],
      compiler_params=pltpu.CompilerParams(use_tc_tiling_on_sc=False,
                                           needs_layout_passes=False),
  )
  def gather_kernel(store_i, cat_i, tags_i, tastes_i,
                    store_t, cat_t, tag_t, taste_t, zpad_i,
                    a_o,
                    traw, sraw, idx_v, r_store, r_cat, r_tag, r_taste,
                    r_tsum, r_ssum, zbuf,
                    sem_i, sem_g, sem_w):
    wid = lax.axis_index("s") * nc + lax.axis_index("c")
    iota16 = lax.iota(jnp.int32, 16)
    pltpu.async_copy(zpad_i, zbuf, sem_i).wait()
    for c in range(cpw):
      r = wid * cpw + c
      base = r * CHUNK
      loads = [
          pltpu.async_copy(store_i.at[r], idx_v.at[0], sem_i),
          pltpu.async_copy(cat_i.at[r], idx_v.at[16], sem_i),
          pltpu.async_copy(tags_i.at[pl.ds(base, CHUNK), :], traw, sem_i),
          pltpu.async_copy(tastes_i.at[pl.ds(base, CHUNK), :], sraw, sem_i),
      ]
      for cp in loads:
        cp.wait()
      for v in range(CHUNK // 16):
        rows = iota16 + (v * 16)
        for j in range(10):
          idx_v[1 + j, pl.ds(v * 16, 16)] = plsc.load_gather(
              traw, [rows, jnp.full((16,), j, jnp.int32)])
        for j in range(5):
          idx_v[11 + j, pl.ds(v * 16, 16)] = plsc.load_gather(
              sraw, [rows, jnp.full((16,), j, jnp.int32)])
      gathers = [
          pltpu.async_copy(store_t.at[idx_v.at[0]], r_store, sem_g),
          pltpu.async_copy(cat_t.at[idx_v.at[16]], r_cat, sem_g),
      ]
      for j in range(10):
        gathers.append(
            pltpu.async_copy(tag_t.at[idx_v.at[1 + j]], r_tag.at[j], sem_g))
      for j in range(5):
        gathers.append(
            pltpu.async_copy(taste_t.at[idx_v.at[11 + j]], r_taste.at[j],
                             sem_g))
      for cp in gathers:
        cp.wait()

      def sum_body(v, carry):
        rows = iota16 + v * 16
        for d in range(de):
          cols = jnp.full((16,), d, jnp.int32)
          acc = plsc.load_gather(r_tag, [jnp.zeros((16,), jnp.int32),
                                         rows, cols])
          for j in range(1, 10):
            acc = acc + plsc.load_gather(
                r_tag, [jnp.full((16,), j, jnp.int32), rows, cols])
          plsc.store_scatter(r_tsum, [rows, cols], acc)
          acc2 = plsc.load_gather(r_taste, [jnp.zeros((16,), jnp.int32),
                                            rows, cols])
          for j in range(1, 5):
            acc2 = acc2 + plsc.load_gather(
                r_taste, [jnp.full((16,), j, jnp.int32), rows, cols])
          plsc.store_scatter(r_ssum, [rows, cols], acc2)
        return carry

      lax.fori_loop(0, CHUNK // 16, sum_body, 0)

      writes = [
          pltpu.async_copy(r_store,
                           a_o.at[pl.ds(base, CHUNK), pl.ds(0, ds_)], sem_w),
          pltpu.async_copy(r_tsum,
                           a_o.at[pl.ds(base, CHUNK), pl.ds(32, de)], sem_w),
          pltpu.async_copy(r_ssum,
                           a_o.at[pl.ds(base, CHUNK), pl.ds(48, de)], sem_w),
          pltpu.async_copy(r_cat,
                           a_o.at[pl.ds(base, CHUNK), pl.ds(64, de)], sem_w),
          pltpu.async_copy(zbuf,
                           a_o.at[pl.ds(base, CHUNK), pl.ds(80, 48)], sem_w),
      ]
      for cp in writes:
        cp.wait()

  return gather_kernel(store2, cat2, tags, tastes,
                       store_table, cat_table, tag_table, taste_table, zpad)


def _tc_body(a1_ref, a2_ref, tags_ref, tastes_ref, day_ref,
             price_ref, ot_ref, rt_ref, loc_ref, tm_ref,
             price_W_ref, price_b_ref, ot_W_ref, ot_b_ref,
             rt_W_ref, rt_b_ref, loc_W_ref, loc_b_ref, tm_W_ref, tm_b_ref,
             day_t_ref, W1a_ref, W1b_ref, b1_ref, W2_ref, b2_ref, W3_ref,
             b3_ref, W1r_ref, out_ref):
  f32 = jnp.float32
  blk = a1_ref.shape[0]

  tmask = (tags_ref[...] != 0).astype(f32)
  tcnt = jnp.sum(tmask, axis=1, keepdims=True)
  smask = (tastes_ref[...] != 0).astype(f32)
  scnt = jnp.sum(smask, axis=1, keepdims=True)
  rt_ = 1.0 / (tcnt + 1e-8)
  rs_ = 1.0 / (scnt + 1e-8)
  lane = lax.broadcasted_iota(jnp.int32, (blk, 128), 1)
  scale = jnp.where(lane < 32, 1.0,
                    jnp.where(lane < 48, rt_,
                              jnp.where(lane < 64, rs_, 1.0)))
  a2 = a2_ref[...] * scale

  price_emb = price_ref[...] * price_W_ref[...] + price_b_ref[...]
  ot_emb = ot_ref[...] * ot_W_ref[...] + ot_b_ref[...]
  rt_emb = rt_ref[...] * rt_W_ref[...] + rt_b_ref[...]
  tm_emb = tm_ref[...] * tm_W_ref[...] + tm_b_ref[...]
  loc_emb = (jnp.dot(loc_ref[...], loc_W_ref[...],
                     preferred_element_type=f32) + loc_b_ref[...])

  iota7 = lax.broadcasted_iota(jnp.int32, (blk, 7), 1)
  day_oh = (day_ref[...] == iota7).astype(f32)
  day_emb = jnp.dot(day_oh, day_t_ref[...], preferred_element_type=f32)

  W1r = W1r_ref[...]
  h = jnp.dot(a1_ref[...], W1a_ref[...], preferred_element_type=f32)
  h = h + jnp.dot(a2, W1b_ref[...], preferred_element_type=f32)
  h = h + jnp.dot(price_emb, W1r[0:16, :], preferred_element_type=f32)
  h = h + jnp.dot(ot_emb, W1r[16:24, :], preferred_element_type=f32)
  h = h + jnp.dot(rt_emb, W1r[24:32, :], preferred_element_type=f32)
  h = h + jnp.dot(loc_emb, W1r[32:48, :], preferred_element_type=f32)
  h = h + jnp.dot(tm_emb, W1r[48:56, :], preferred_element_type=f32)
  h = h + jnp.dot(day_emb, W1r[56:64, :], preferred_element_type=f32)
  h = jnp.maximum(h + b1_ref[...], 0.0)

  h = jnp.dot(h, W2_ref[...], preferred_element_type=f32) + b2_ref[...]
  h = jnp.maximum(h, 0.0)
  out = jnp.dot(h, W3_ref[...], preferred_element_type=f32) + b3_ref[...]

  nrm = jnp.sqrt(jnp.sum(out * out, axis=-1, keepdims=True))
  out_ref[...] = out / jnp.maximum(nrm, 1e-12)


def kernel(dish_id, store_id, tags, tastes, category, price, order_times,
           rating, location, time_of_day, day_of_week, dish_table,
           store_table, tag_table, taste_table, cat_table, day_table,
           price_W, price_b, ot_W, ot_b, rt_W, rt_b, loc_W, loc_b, tm_W,
           tm_b, W1, b1, W2, b2, W3, b3):
  B = dish_id.shape[0]
  nb = B // CHUNK

  i32 = jnp.int32
  dish2 = dish_id.astype(i32).reshape(nb, CHUNK)
  store2 = store_id.astype(i32).reshape(nb, CHUNK)

  tag_tz = tag_table.at[0].set(0.0)
  taste_tz = taste_table.at[0].set(0.0)

  zpad = jnp.zeros((CHUNK, 64), jnp.float32)
  zpad2 = jnp.zeros((CHUNK, 48), jnp.float32)
  cat2 = category.astype(i32).reshape(nb, CHUNK)
  a1 = _sc_dish(dish2, dish_table, zpad)
  a2 = _sc_gather(store2, cat2, tags.astype(i32), tastes.astype(i32),
                  store_table, cat_table, tag_tz, taste_tz, zpad2)

  W1a = jnp.concatenate([W1[0:64], jnp.zeros((64, 128), jnp.float32)], 0)
  W1b = jnp.concatenate([W1[64:144], jnp.zeros((48, 128), jnp.float32)], 0)

  BLK = 2048
  grid = (B // BLK,)

  def row_spec(w):
    return pl.BlockSpec((BLK, w), lambda i: (i, 0))

  def full_spec(shape):
    nd = len(shape)
    return pl.BlockSpec(shape, lambda i: (0,) * nd)

  out = pl.pallas_call(
      _tc_body,
      grid=grid,
      in_specs=[
          row_spec(128), row_spec(128),
          row_spec(10), row_spec(5), row_spec(1),
          row_spec(1), row_spec(1), row_spec(1), row_spec(2), row_spec(1),
          full_spec((1, 16)), full_spec((1, 16)),
          full_spec((1, 8)), full_spec((1, 8)),
          full_spec((1, 8)), full_spec((1, 8)),
          full_spec((2, 16)), full_spec((1, 16)),
          full_spec((1, 8)), full_spec((1, 8)),
          full_spec((7, 8)),
          full_spec((128, 128)), full_spec((128, 128)),
          full_spec((1, 128)),
          full_spec((128, 64)), full_spec((1, 64)),
          full_spec((64, 64)), full_spec((1, 64)),
          full_spec((64, 128)),
      ],
      out_specs=row_spec(64),
      out_shape=jax.ShapeDtypeStruct((B, 64), jnp.float32),
      compiler_params=pltpu.CompilerParams(
          dimension_semantics=("parallel",)),
  )(
      a1, a2,
      tags.astype(i32), tastes.astype(i32),
      day_of_week.astype(i32).reshape(B, 1),
      price, order_times, rating, location, time_of_day,
      price_W, price_b.reshape(1, 16), ot_W, ot_b.reshape(1, 8),
      rt_W, rt_b.reshape(1, 8), loc_W, loc_b.reshape(1, 16),
      tm_W, tm_b.reshape(1, 8), day_table,
      W1a, W1b, b1.reshape(1, 128), W2, b2.reshape(1, 64),
      W3, b3.reshape(1, 64), W1[144:208],
  )
  return out
